# baseline (speedup 1.0000x reference)
import functools

import jax
import jax.numpy as jnp
import numpy as np
from jax import lax
from jax.experimental import pallas as pl
from jax.experimental.pallas import tpu as pltpu
from jax.experimental.pallas import tpu_sc as plsc

_NC = 2
_NS = 16
_NW = _NC * _NS

_N = 10000
_E = 160000
_D = 128
_T = 2
_RING_DIM = 10
_HID = 128


_A_ROWS = 1000


def _proj_body(x_ref, w_ref, p0_ref, p1_ref, p2_ref, rp_ref):
    acc = jnp.dot(x_ref[...], w_ref[...], preferred_element_type=jnp.float32)
    p0_ref[...] = acc[:, 0:128]
    p1_ref[...] = acc[:, 128:256]
    p2_ref[...] = acc[:, 256:384]
    rp_ref[...] = acc[:, 384:512]


def _project_tables(x, wall):
    n = x.shape[0]
    grid = n // _A_ROWS
    out = jax.ShapeDtypeStruct((n, 128), jnp.float32)
    return pl.pallas_call(
        _proj_body,
        grid=(grid,),
        in_specs=[
            pl.BlockSpec((_A_ROWS, 144), lambda i: (i, 0)),
            pl.BlockSpec((144, 512), lambda i: (0, 0)),
        ],
        out_specs=[pl.BlockSpec((_A_ROWS, 128), lambda i: (i, 0))] * 4,
        out_shape=[out, out, out, out],
    )(x, wall)


_GCH = 64
_GTASKS = _E // _GCH
_GIT = -(-_GTASKS // _NW)


def _gather_chunk_start(idx4, p0, p1, p2, rp, bufs, sem):
    i0, i1, i2, i3, g0, g1, g2, g3 = bufs
    lanes4 = lax.iota(jnp.int32, 16) * 4
    for g in range(_GCH // 16):
        off = g * 16
        sl = pl.ds(off, 16)
        i0[sl] = plsc.load_gather(idx4, [lanes4 + (off * 4 + 0)])
        i1[sl] = plsc.load_gather(idx4, [lanes4 + (off * 4 + 1)])
        i2[sl] = plsc.load_gather(idx4, [lanes4 + (off * 4 + 2)])
        i3[sl] = plsc.load_gather(idx4, [lanes4 + (off * 4 + 3)])
    return (pltpu.async_copy(p0.at[i0], g0, sem),
            pltpu.async_copy(p1.at[i1], g1, sem),
            pltpu.async_copy(p2.at[i2], g2, sem),
            pltpu.async_copy(rp.at[i3], g3, sem))


def _gather_chunk_finish(q, descs, bufs, cv, dr, s_out, dr_out):
    i0, i1, i2, i3, g0, g1, g2, g3 = bufs
    base = q * _GCH
    lanes = lax.iota(jnp.int32, 16)
    for g in range(_GCH // 16):
        off = g * 16
        n0 = i0[pl.ds(off, 16)]
        n2 = i2[pl.ds(off, 16)]
        rows = (lanes + off) * 16
        for k in range(3):
            x0 = plsc.load_gather(cv, [n0 + k * _N])
            x2 = plsc.load_gather(cv, [n2 + k * _N])
            plsc.store_scatter(dr, [rows + k], x0 - x2)
    pltpu.sync_copy(dr, dr_out.at[pl.ds(base * 16, _GCH * 16)])
    for d in descs:
        d.wait()

    def row(i, _):
        for j in range(8):
            sl = pl.ds(j * 16, 16)
            g0[i, sl] = g0[i, sl] + g1[i, sl] + g2[i, sl] + g3[i, sl]
        return 0

    lax.fori_loop(0, _GCH, row, 0, unroll=False)
    pltpu.sync_copy(g0, s_out.at[pl.ds(base, _GCH)])


def _make_gather_body(ntasks):
    nit = -(-ntasks // _NW)

    def body(p0, p1, p2, rp, coords_t, angles4,
             s_out, dr_out,
             ia, ib, bufa, bufb, cv, dr, sem, isem):
        wid = lax.axis_index("s") * _NC + lax.axis_index("c")

        pltpu.sync_copy(coords_t, cv)

        def zrow(i, _):
            dr[pl.ds(i * 16, 16)] = jnp.zeros((16,), jnp.float32)
            return 0

        lax.fori_loop(0, _GCH, zrow, 0, unroll=False)

        def qof(it):
            return wid + _NW * it

        def idx_load(q, iref):
            return pltpu.async_copy(
                angles4.at[pl.ds(q * (_GCH * 4), _GCH * 4)], iref, isem)

        da = idx_load(qof(0), ia)

        def pair(k, _):
            it0 = 2 * k

            @pl.when(qof(it0) < ntasks)
            def _():
                da.wait()

                @pl.when(qof(it0 + 1) < ntasks)
                def _():
                    idx_load(qof(it0 + 1), ib).wait()

                dsc_a = _gather_chunk_start(ia, p0, p1, p2, rp, bufa, sem)

                @pl.when(qof(it0 + 1) < ntasks)
                def _():
                    dsc_b = _gather_chunk_start(ib, p0, p1, p2, rp, bufb, sem)

                    @pl.when(qof(it0 + 2) < ntasks)
                    def _():
                        idx_load(qof(it0 + 2), ia)

                    _gather_chunk_finish(qof(it0), dsc_a, bufa, cv, dr,
                                         s_out, dr_out)
                    _gather_chunk_finish(qof(it0 + 1), dsc_b, bufb, cv, dr,
                                         s_out, dr_out)

                @pl.when(qof(it0 + 1) >= ntasks)
                def _():
                    _gather_chunk_finish(qof(it0), dsc_a, bufa, cv, dr,
                                         s_out, dr_out)

            return 0

        lax.fori_loop(0, -(-nit // 2), pair, 0, unroll=False)

    return body


def _gather_stage(p0, p1, p2, rp, coords_t, angles4):
    mesh = plsc.VectorSubcoreMesh(core_axis_name="c", subcore_axis_name="s")
    f32 = jnp.float32
    i32 = jnp.int32
    ne = angles4.shape[0] // 4

    def gbufs():
        return ([pltpu.VMEM((_GCH,), i32) for _ in range(4)]
                + [pltpu.VMEM((_GCH, 128), f32) for _ in range(4)])

    kern = functools.partial(
        pl.kernel,
        out_type=[
            jax.ShapeDtypeStruct((ne, 128), f32),
            jax.ShapeDtypeStruct((ne * 16,), f32),
        ],
        mesh=mesh,
        scratch_types=[
            pltpu.VMEM((_GCH * 4,), i32),
            pltpu.VMEM((_GCH * 4,), i32),
            gbufs(),
            gbufs(),
            pltpu.VMEM((3 * _N,), f32),
            pltpu.VMEM((_GCH * 16,), f32),
            pltpu.SemaphoreType.DMA,
            pltpu.SemaphoreType.DMA,
        ],
        compiler_params=pltpu.CompilerParams(needs_layout_passes=False),
    )(_make_gather_body(ne // _GCH))
    return kern(p0, p1, p2, rp, coords_t, angles4)


_BE = 2000


def _leaky(x):
    return jnp.where(x >= 0, x, 0.001 * x)


def _dot3(h, w_hi, w_lo):
    bf16 = jnp.bfloat16
    f32 = jnp.float32
    h_hi = h.astype(bf16)
    h_lo = (h - h_hi.astype(f32)).astype(bf16)
    return (jnp.dot(h_hi, w_hi, preferred_element_type=f32)
            + jnp.dot(h_hi, w_lo, preferred_element_type=f32)
            + jnp.dot(h_lo, w_hi, preferred_element_type=f32))


def _mlp_body(s_ref, dr_ref, b0t_ref, wdl_ref, w1h_ref, w1l_ref, b1_ref,
              w2h_ref, w2l_ref, b2_ref, w3_ref, b3_ref, pay_ref):
    s = s_ref[...]
    dr = dr_ref[...]
    dl2 = jnp.sum(dr * dr, axis=1, keepdims=True)
    dl = jnp.sqrt(jnp.maximum(dl2, 1e-12))
    dh = dr * (1.0 / dl)
    base = s + dl * wdl_ref[...]
    u = []
    for ti in range(_T):
        h = _leaky(base + b0t_ref[ti:ti + 1, :])
        h = _leaky(_dot3(h, w1h_ref[...], w1l_ref[...]) + b1_ref[...])
        h = _leaky(_dot3(h, w2h_ref[...], w2l_ref[...]) + b2_ref[...])
        d = (jnp.dot(h, w3_ref[...],
                     preferred_element_type=jnp.float32) + b3_ref[...])
        u0 = (-0.5 * d[:, 0:1]) * dh
        u2 = (0.5 * d[:, 1:2]) * dh
        u.append((u0, u2))
    pay_ref[0] = jnp.concatenate([u[0][0][:, 0:4], u[1][0][:, 0:4]], axis=1)
    pay_ref[1] = jnp.concatenate([u[0][1][:, 0:4], u[1][1][:, 0:4]], axis=1)


def _mlp_stage(s, dr, b0t, wdl, w1h, w1l, b1, w2h, w2l, b2, w3p, b3p):
    ne = s.shape[0]
    return pl.pallas_call(
        _mlp_body,
        grid=(ne // _BE,),
        in_specs=[
            pl.BlockSpec((_BE, 128), lambda i: (i, 0)),
            pl.BlockSpec((_BE, 16), lambda i: (i, 0)),
            pl.BlockSpec((2, 128), lambda i: (0, 0)),
            pl.BlockSpec((1, 128), lambda i: (0, 0)),
            pl.BlockSpec((128, 128), lambda i: (0, 0)),
            pl.BlockSpec((128, 128), lambda i: (0, 0)),
            pl.BlockSpec((1, 128), lambda i: (0, 0)),
            pl.BlockSpec((128, 128), lambda i: (0, 0)),
            pl.BlockSpec((128, 128), lambda i: (0, 0)),
            pl.BlockSpec((1, 128), lambda i: (0, 0)),
            pl.BlockSpec((128, 8), lambda i: (0, 0)),
            pl.BlockSpec((1, 8), lambda i: (0, 0)),
        ],
        out_specs=pl.BlockSpec((2, _BE, 8), lambda i: (0, i, 0)),
        out_shape=jax.ShapeDtypeStruct((2, ne, 8), jnp.float32),
    )(s, dr, b0t, wdl, w1h, w1l, b1, w2h, w2l, b2, w3p, b3p)


_ACC = _N * 8
_ZLEN = _ACC // _NS


def _make_scatter_body(ntasks):
    def body(pay2d, idx2d, out, idx_v, pay_v, stage_v, acc, lsem, ssem):
        c = lax.axis_index("c")
        s = lax.axis_index("s")
        wid = s * _NC + c

        def zvec(i, _):
            stage_v[pl.ds(i * 16, 16)] = jnp.zeros((16,), jnp.float32)
            return 0

        lax.fori_loop(0, _ZLEN // 16, zvec, 0, unroll=False)
        if _ZLEN % 16:
            stage_v[pl.ds(_ZLEN - 16, 16)] = jnp.zeros((16,), jnp.float32)
        pltpu.sync_copy(stage_v, acc.at[pl.ds(s * _ZLEN, _ZLEN)])
        plsc.subcore_barrier()

        def task(it, _):
            q = wid + _NW * it

            @pl.when(q < ntasks)
            def _():
                r = q * 8
                di = pltpu.async_copy(idx2d.at[pl.ds(r, 8)], idx_v, lsem)
                dp = pltpu.async_copy(pay2d.at[pl.ds(r, 8)], pay_v, lsem)
                di.wait()
                dp.wait()
                descs = [pltpu.async_copy(pay_v.at[j], acc.at[idx_v.at[j]],
                                          ssem, add=True) for j in range(8)]
                for d in descs:
                    d.wait()

            return 0

        lax.fori_loop(0, (ntasks + _NW - 1) // _NW, task, 0, unroll=False)
        plsc.subcore_barrier()
        pltpu.sync_copy(acc.at[pl.ds(s * _ZLEN, _ZLEN)], stage_v)
        pltpu.sync_copy(stage_v, out.at[pl.ds(c * _ACC + s * _ZLEN, _ZLEN)])

    return body


def _scatter_stage(pay2d, idx2d):
    mesh = plsc.VectorSubcoreMesh(core_axis_name="c", subcore_axis_name="s")
    f32 = jnp.float32
    kern = functools.partial(
        pl.kernel,
        out_type=jax.ShapeDtypeStruct((_NC * _ACC,), f32),
        mesh=mesh,
        scratch_types=[
            pltpu.VMEM((8, 128), jnp.int32),
            pltpu.VMEM((8, 128), f32),
            pltpu.VMEM((_ZLEN,), f32),
            pltpu.VMEM_SHARED((_ACC,), f32),
            pltpu.SemaphoreType.DMA,
            pltpu.SemaphoreType.DMA,
        ],
        compiler_params=pltpu.CompilerParams(needs_layout_passes=False),
    )(_make_scatter_body(pay2d.shape[0] // 8))
    return kern(pay2d, idx2d)


def _combine_body(x_ref, a_ref, o_ref):
    acc = a_ref[...]
    for i in range(x_ref.shape[0]):
        acc = acc + x_ref[i]
    o_ref[...] = acc


def _combine_stage(partials, ans_pad):
    return pl.pallas_call(
        _combine_body,
        out_shape=jax.ShapeDtypeStruct((625, 128), jnp.float32),
    )(partials, ans_pad)


def kernel(coords, angles, encoded, t, answer, ring_table,
           W0, b0, W1, b1, W2, b2, W3, b3):
    assert coords.shape == (_N, 3) and angles.shape == (_E, 4)
    ring_norm = 1.0 / np.sqrt(float(_RING_DIM))

    x = jnp.concatenate(
        [encoded, jnp.pad(ring_table, ((0, 0), (0, 16 - _RING_DIM)))], axis=1)
    wall = jnp.zeros((144, 512), jnp.float32)
    wall = wall.at[0:128, 0:384].set(W0[0:384].reshape(3, 128, 128)
                                     .transpose(1, 0, 2).reshape(128, 384))
    wall = wall.at[128:128 + _RING_DIM, 384:512].set(W0[384:384 + _RING_DIM] * ring_norm)
    coords_t = coords.T.reshape(3 * _N)
    b0t = b0[None, :] + t[:, None] * W0[384 + _RING_DIM][None, :]
    wdl = W0[384 + _RING_DIM + 1][None, :]
    w3p = jnp.pad(W3, ((0, 0), (0, 6)))
    b3p = jnp.pad(b3, (0, 6))[None, :]
    bf16 = jnp.bfloat16
    w1h = W1.astype(bf16)
    w1l = (W1 - w1h.astype(jnp.float32)).astype(bf16)
    w2h = W2.astype(bf16)
    w2l = (W2 - w2h.astype(jnp.float32)).astype(bf16)

    a0 = angles[:, 0]
    a2 = angles[:, 2]
    angles4 = angles.reshape(_E * 4)

    lanes8 = jnp.arange(8, dtype=jnp.int32)
    idx = jnp.stack([a0, a2]) * 8
    idx3 = idx[:, :, None] + lanes8

    p0, p1, p2, rp = _project_tables(x, wall)
    s, dr = _gather_stage(p0, p1, p2, rp, coords_t, angles4)
    dr = dr.reshape(_E, 16)
    pay = _mlp_stage(s, dr, b0t, wdl, w1h, w1l, b1[None, :],
                     w2h, w2l, b2[None, :], w3p, b3p)
    srows = 2 * _E * 8 // 128
    idx2d = idx3.reshape(srows, 128)
    part = _scatter_stage(pay.reshape(srows, 128), idx2d)

    ans_pad = jnp.pad(answer, ((0, 0), (0, 0), (0, 1))).reshape(625, 128)
    comb = _combine_stage(part.reshape(_NC, 625, 128), ans_pad)
    return comb.reshape(_N, _T, 4)[:, :, :3]


# speedup vs baseline: 1.2271x; 1.2271x over previous
import functools

import jax
import jax.numpy as jnp
import numpy as np
from jax import lax
from jax.experimental import pallas as pl
from jax.experimental.pallas import tpu as pltpu
from jax.experimental.pallas import tpu_sc as plsc

_NC = 2
_NS = 16
_NW = _NC * _NS

_N = 10000
_E = 160000
_D = 128
_T = 2
_RING_DIM = 10
_HID = 128


_A_ROWS = 1000


def _proj_body(x_ref, w_ref, p0_ref, p1_ref, p2_ref, rp_ref):
    acc = jnp.dot(x_ref[...], w_ref[...], preferred_element_type=jnp.float32)
    p0_ref[...] = acc[:, 0:128]
    p1_ref[...] = acc[:, 128:256]
    p2_ref[...] = acc[:, 256:384]
    rp_ref[...] = acc[:, 384:512]


def _project_tables(x, wall):
    n = x.shape[0]
    grid = n // _A_ROWS
    out = jax.ShapeDtypeStruct((n, 128), jnp.float32)
    return pl.pallas_call(
        _proj_body,
        grid=(grid,),
        in_specs=[
            pl.BlockSpec((_A_ROWS, 144), lambda i: (i, 0)),
            pl.BlockSpec((144, 512), lambda i: (0, 0)),
        ],
        out_specs=[pl.BlockSpec((_A_ROWS, 128), lambda i: (i, 0))] * 4,
        out_shape=[out, out, out, out],
    )(x, wall)


_GCH = 64
_GTASKS = _E // _GCH
_GIT = -(-_GTASKS // _NW)


def _gather_chunk_start(idx4, p0, p1, p2, rp, bufs, sem):
    i0, i1, i2, i3, g0, g1, g2, g3 = bufs
    lanes4 = lax.iota(jnp.int32, 16) * 4
    for g in range(_GCH // 16):
        off = g * 16
        sl = pl.ds(off, 16)
        i0[sl] = plsc.load_gather(idx4, [lanes4 + (off * 4 + 0)])
        i1[sl] = plsc.load_gather(idx4, [lanes4 + (off * 4 + 1)])
        i2[sl] = plsc.load_gather(idx4, [lanes4 + (off * 4 + 2)])
        i3[sl] = plsc.load_gather(idx4, [lanes4 + (off * 4 + 3)])
    return (pltpu.async_copy(p0.at[i0], g0, sem),
            pltpu.async_copy(p1.at[i1], g1, sem),
            pltpu.async_copy(p2.at[i2], g2, sem),
            pltpu.async_copy(rp.at[i3], g3, sem))


def _gather_chunk_finish(q, descs, bufs, cv, dr, s_out, dr_out):
    i0, i1, i2, i3, g0, g1, g2, g3 = bufs
    base = q * _GCH
    lanes = lax.iota(jnp.int32, 16)
    for g in range(_GCH // 16):
        off = g * 16
        n0 = i0[pl.ds(off, 16)]
        n2 = i2[pl.ds(off, 16)]
        rows = (lanes + off) * 16
        for k in range(3):
            x0 = plsc.load_gather(cv, [n0 + k * _N])
            x2 = plsc.load_gather(cv, [n2 + k * _N])
            plsc.store_scatter(dr, [rows + k], x0 - x2)
    pltpu.sync_copy(dr, dr_out.at[pl.ds(base * 16, _GCH * 16)])
    for d in descs:
        d.wait()

    def row(i, _):
        for j in range(8):
            sl = pl.ds(j * 16, 16)
            g0[i, sl] = g0[i, sl] + g1[i, sl] + g2[i, sl] + g3[i, sl]
        return 0

    lax.fori_loop(0, _GCH, row, 0, unroll=False)
    pltpu.sync_copy(g0, s_out.at[pl.ds(base, _GCH)])


def _make_gather_body(ntasks):
    nit = -(-ntasks // _NW)

    def body(p0, p1, p2, rp, coords_t, angles4,
             s_out, dr_out,
             ia, ib, bufa, bufb, cv, dr, sem, isem):
        wid = lax.axis_index("s") * _NC + lax.axis_index("c")

        pltpu.sync_copy(coords_t, cv)

        def zrow(i, _):
            dr[pl.ds(i * 16, 16)] = jnp.zeros((16,), jnp.float32)
            return 0

        lax.fori_loop(0, _GCH, zrow, 0, unroll=False)

        def qof(it):
            return wid + _NW * it

        def idx_load(q, iref):
            return pltpu.async_copy(
                angles4.at[pl.ds(q * (_GCH * 4), _GCH * 4)], iref, isem)

        da = idx_load(qof(0), ia)

        def pair(k, _):
            it0 = 2 * k

            @pl.when(qof(it0) < ntasks)
            def _():
                da.wait()

                @pl.when(qof(it0 + 1) < ntasks)
                def _():
                    idx_load(qof(it0 + 1), ib).wait()

                dsc_a = _gather_chunk_start(ia, p0, p1, p2, rp, bufa, sem)

                @pl.when(qof(it0 + 1) < ntasks)
                def _():
                    dsc_b = _gather_chunk_start(ib, p0, p1, p2, rp, bufb, sem)

                    @pl.when(qof(it0 + 2) < ntasks)
                    def _():
                        idx_load(qof(it0 + 2), ia)

                    _gather_chunk_finish(qof(it0), dsc_a, bufa, cv, dr,
                                         s_out, dr_out)
                    _gather_chunk_finish(qof(it0 + 1), dsc_b, bufb, cv, dr,
                                         s_out, dr_out)

                @pl.when(qof(it0 + 1) >= ntasks)
                def _():
                    _gather_chunk_finish(qof(it0), dsc_a, bufa, cv, dr,
                                         s_out, dr_out)

            return 0

        lax.fori_loop(0, -(-nit // 2), pair, 0, unroll=False)

    return body


def _gather_stage(p0, p1, p2, rp, coords_t, angles4):
    mesh = plsc.VectorSubcoreMesh(core_axis_name="c", subcore_axis_name="s")
    f32 = jnp.float32
    i32 = jnp.int32
    ne = angles4.shape[0] // 4

    def gbufs():
        return ([pltpu.VMEM((_GCH,), i32) for _ in range(4)]
                + [pltpu.VMEM((_GCH, 128), f32) for _ in range(4)])

    kern = functools.partial(
        pl.kernel,
        out_type=[
            jax.ShapeDtypeStruct((ne, 128), f32),
            jax.ShapeDtypeStruct((ne * 16,), f32),
        ],
        mesh=mesh,
        scratch_types=[
            pltpu.VMEM((_GCH * 4,), i32),
            pltpu.VMEM((_GCH * 4,), i32),
            gbufs(),
            gbufs(),
            pltpu.VMEM((3 * _N,), f32),
            pltpu.VMEM((_GCH * 16,), f32),
            pltpu.SemaphoreType.DMA,
            pltpu.SemaphoreType.DMA,
        ],
        compiler_params=pltpu.CompilerParams(needs_layout_passes=False),
    )(_make_gather_body(ne // _GCH))
    return kern(p0, p1, p2, rp, coords_t, angles4)


_BE = 2000


def _leaky(x):
    return jnp.where(x >= 0, x, 0.001 * x)


def _mlp_body(s_ref, dr_ref, b0t_ref, wdl_ref, w1h_ref, b1_ref,
              w2h_ref, b2_ref, w3_ref, b3_ref, pay_ref):
    s = s_ref[...]
    dr = dr_ref[...]
    dl2 = jnp.sum(dr * dr, axis=1, keepdims=True)
    dl = jnp.sqrt(jnp.maximum(dl2, 1e-12))
    dh = dr * (1.0 / dl)
    base = s + dl * wdl_ref[...]
    u = []
    for ti in range(_T):
        h = _leaky(base + b0t_ref[ti:ti + 1, :])
        h = _leaky(jnp.dot(h, w1h_ref[...],
                           preferred_element_type=jnp.float32) + b1_ref[...])
        h = _leaky(jnp.dot(h, w2h_ref[...],
                           preferred_element_type=jnp.float32) + b2_ref[...])
        d = (jnp.dot(h, w3_ref[...],
                     preferred_element_type=jnp.float32) + b3_ref[...])
        u0 = (-0.5 * d[:, 0:1]) * dh
        u2 = (0.5 * d[:, 1:2]) * dh
        u.append((u0, u2))
    pay_ref[0] = jnp.concatenate([u[0][0][:, 0:4], u[1][0][:, 0:4]], axis=1)
    pay_ref[1] = jnp.concatenate([u[0][1][:, 0:4], u[1][1][:, 0:4]], axis=1)


def _mlp_stage(s, dr, b0t, wdl, w1, b1, w2, b2, w3p, b3p):
    ne = s.shape[0]
    return pl.pallas_call(
        _mlp_body,
        grid=(ne // _BE,),
        in_specs=[
            pl.BlockSpec((_BE, 128), lambda i: (i, 0)),
            pl.BlockSpec((_BE, 16), lambda i: (i, 0)),
            pl.BlockSpec((2, 128), lambda i: (0, 0)),
            pl.BlockSpec((1, 128), lambda i: (0, 0)),
            pl.BlockSpec((128, 128), lambda i: (0, 0)),
            pl.BlockSpec((1, 128), lambda i: (0, 0)),
            pl.BlockSpec((128, 128), lambda i: (0, 0)),
            pl.BlockSpec((1, 128), lambda i: (0, 0)),
            pl.BlockSpec((128, 8), lambda i: (0, 0)),
            pl.BlockSpec((1, 8), lambda i: (0, 0)),
        ],
        out_specs=pl.BlockSpec((2, _BE, 8), lambda i: (0, i, 0)),
        out_shape=jax.ShapeDtypeStruct((2, ne, 8), jnp.float32),
    )(s, dr, b0t, wdl, w1, b1, w2, b2, w3p, b3p)


_ACC = _N * 8
_ZLEN = _ACC // _NS


def _make_scatter_body(ntasks):
    def body(pay2d, idx2d, out, idx_v, pay_v, stage_v, acc, lsem, ssem):
        c = lax.axis_index("c")
        s = lax.axis_index("s")
        wid = s * _NC + c

        def zvec(i, _):
            stage_v[pl.ds(i * 16, 16)] = jnp.zeros((16,), jnp.float32)
            return 0

        lax.fori_loop(0, _ZLEN // 16, zvec, 0, unroll=False)
        if _ZLEN % 16:
            stage_v[pl.ds(_ZLEN - 16, 16)] = jnp.zeros((16,), jnp.float32)
        pltpu.sync_copy(stage_v, acc.at[pl.ds(s * _ZLEN, _ZLEN)])
        plsc.subcore_barrier()

        def task(it, _):
            q = wid + _NW * it

            @pl.when(q < ntasks)
            def _():
                r = q * 8
                di = pltpu.async_copy(idx2d.at[pl.ds(r, 8)], idx_v, lsem)
                dp = pltpu.async_copy(pay2d.at[pl.ds(r, 8)], pay_v, lsem)
                di.wait()
                dp.wait()
                descs = [pltpu.async_copy(pay_v.at[j], acc.at[idx_v.at[j]],
                                          ssem, add=True) for j in range(8)]
                for d in descs:
                    d.wait()

            return 0

        lax.fori_loop(0, (ntasks + _NW - 1) // _NW, task, 0, unroll=False)
        plsc.subcore_barrier()
        pltpu.sync_copy(acc.at[pl.ds(s * _ZLEN, _ZLEN)], stage_v)
        pltpu.sync_copy(stage_v, out.at[pl.ds(c * _ACC + s * _ZLEN, _ZLEN)])

    return body


def _scatter_stage(pay2d, idx2d):
    mesh = plsc.VectorSubcoreMesh(core_axis_name="c", subcore_axis_name="s")
    f32 = jnp.float32
    kern = functools.partial(
        pl.kernel,
        out_type=jax.ShapeDtypeStruct((_NC * _ACC,), f32),
        mesh=mesh,
        scratch_types=[
            pltpu.VMEM((8, 128), jnp.int32),
            pltpu.VMEM((8, 128), f32),
            pltpu.VMEM((_ZLEN,), f32),
            pltpu.VMEM_SHARED((_ACC,), f32),
            pltpu.SemaphoreType.DMA,
            pltpu.SemaphoreType.DMA,
        ],
        compiler_params=pltpu.CompilerParams(needs_layout_passes=False),
    )(_make_scatter_body(pay2d.shape[0] // 8))
    return kern(pay2d, idx2d)


def _combine_body(x_ref, a_ref, o_ref):
    acc = a_ref[...]
    for i in range(x_ref.shape[0]):
        acc = acc + x_ref[i]
    o_ref[...] = acc


def _combine_stage(partials, ans_pad):
    return pl.pallas_call(
        _combine_body,
        out_shape=jax.ShapeDtypeStruct((625, 128), jnp.float32),
    )(partials, ans_pad)


def kernel(coords, angles, encoded, t, answer, ring_table,
           W0, b0, W1, b1, W2, b2, W3, b3):
    assert coords.shape == (_N, 3) and angles.shape == (_E, 4)
    ring_norm = 1.0 / np.sqrt(float(_RING_DIM))

    x = jnp.concatenate(
        [encoded, jnp.pad(ring_table, ((0, 0), (0, 16 - _RING_DIM)))], axis=1)
    wall = jnp.zeros((144, 512), jnp.float32)
    wall = wall.at[0:128, 0:384].set(W0[0:384].reshape(3, 128, 128)
                                     .transpose(1, 0, 2).reshape(128, 384))
    wall = wall.at[128:128 + _RING_DIM, 384:512].set(W0[384:384 + _RING_DIM] * ring_norm)
    coords_t = coords.T.reshape(3 * _N)
    b0t = b0[None, :] + t[:, None] * W0[384 + _RING_DIM][None, :]
    wdl = W0[384 + _RING_DIM + 1][None, :]
    w3p = jnp.pad(W3, ((0, 0), (0, 6)))
    b3p = jnp.pad(b3, (0, 6))[None, :]

    a0 = angles[:, 0]
    a2 = angles[:, 2]
    angles4 = angles.reshape(_E * 4)

    lanes8 = jnp.arange(8, dtype=jnp.int32)
    idx = jnp.stack([a0, a2]) * 8
    idx3 = idx[:, :, None] + lanes8

    p0, p1, p2, rp = _project_tables(x, wall)
    s, dr = _gather_stage(p0, p1, p2, rp, coords_t, angles4)
    dr = dr.reshape(_E, 16)
    pay = _mlp_stage(s, dr, b0t, wdl, W1, b1[None, :], W2, b2[None, :],
                     w3p, b3p)
    srows = 2 * _E * 8 // 128
    idx2d = idx3.reshape(srows, 128)
    part = _scatter_stage(pay.reshape(srows, 128), idx2d)

    ans_pad = jnp.pad(answer, ((0, 0), (0, 0), (0, 1))).reshape(625, 128)
    comb = _combine_stage(part.reshape(_NC, 625, 128), ans_pad)
    return comb.reshape(_N, _T, 4)[:, :, :3]


# speedup vs baseline: 1.3734x; 1.1193x over previous
import functools

import jax
import jax.numpy as jnp
import numpy as np
from jax import lax
from jax.experimental import pallas as pl
from jax.experimental.pallas import tpu as pltpu
from jax.experimental.pallas import tpu_sc as plsc

_NC = 2
_NS = 16
_NW = _NC * _NS

_N = 10000
_E = 160000
_D = 128
_T = 2
_RING_DIM = 10
_HID = 128


_A_ROWS = 1000


def _proj_body(x_ref, w_ref, p0_ref, p1_ref, p2_ref, rp_ref):
    acc = jnp.dot(x_ref[...], w_ref[...], preferred_element_type=jnp.float32)
    p0_ref[...] = acc[:, 0:128]
    p1_ref[...] = acc[:, 128:256]
    p2_ref[...] = acc[:, 256:384]
    rp_ref[...] = acc[:, 384:512]


def _project_tables(x, wall):
    n = x.shape[0]
    grid = n // _A_ROWS
    out = jax.ShapeDtypeStruct((n, 128), jnp.float32)
    return pl.pallas_call(
        _proj_body,
        grid=(grid,),
        in_specs=[
            pl.BlockSpec((_A_ROWS, 144), lambda i: (i, 0)),
            pl.BlockSpec((144, 512), lambda i: (0, 0)),
        ],
        out_specs=[pl.BlockSpec((_A_ROWS, 128), lambda i: (i, 0))] * 4,
        out_shape=[out, out, out, out],
    )(x, wall)


_GCH = 64
_GTASKS = _E // _GCH
_GIT = -(-_GTASKS // _NW)


def _gather_chunk_start(p0, p1, p2, rp, bufs, sem):
    i0, i1, i2, i3, g0, g1, g2, g3 = bufs
    return (pltpu.async_copy(p0.at[i0], g0, sem),
            pltpu.async_copy(p1.at[i1], g1, sem),
            pltpu.async_copy(p2.at[i2], g2, sem),
            pltpu.async_copy(rp.at[i3], g3, sem))


def _gather_chunk_finish(q, descs, bufs, cv, dr, s_out, dr_out):
    i0, i1, i2, i3, g0, g1, g2, g3 = bufs
    base = q * _GCH
    lanes = lax.iota(jnp.int32, 16)
    for g in range(_GCH // 16):
        off = g * 16
        n0 = i0[pl.ds(off, 16)]
        n2 = i2[pl.ds(off, 16)]
        rows = (lanes + off) * 16
        for k in range(3):
            x0 = plsc.load_gather(cv, [n0 + k * _N])
            x2 = plsc.load_gather(cv, [n2 + k * _N])
            plsc.store_scatter(dr, [rows + k], x0 - x2)
    pltpu.sync_copy(dr, dr_out.at[pl.ds(base * 16, _GCH * 16)])
    for d in descs:
        d.wait()

    def row(i, _):
        for j in range(8):
            sl = pl.ds(j * 16, 16)
            g0[i, sl] = g0[i, sl] + g1[i, sl] + g2[i, sl] + g3[i, sl]
        return 0

    lax.fori_loop(0, _GCH, row, 0, unroll=False)
    pltpu.sync_copy(g0, s_out.at[pl.ds(base, _GCH)])


def _make_gather_body(ne):
    ntasks = ne // _GCH
    nit = -(-ntasks // _NW)

    def body(p0, p1, p2, rp, coords_t, cat4,
             s_out, dr_out,
             bufa, bufb, cv, dr, sem, isem):
        wid = lax.axis_index("s") * _NC + lax.axis_index("c")

        pltpu.sync_copy(coords_t, cv)

        def zrow(i, _):
            dr[pl.ds(i * 16, 16)] = jnp.zeros((16,), jnp.float32)
            return 0

        lax.fori_loop(0, _GCH, zrow, 0, unroll=False)

        def qof(it):
            return wid + _NW * it

        def idx_load(q, bufs):
            return [pltpu.async_copy(
                cat4.at[pl.ds(k * ne + q * _GCH, _GCH)], bufs[k], isem)
                for k in range(4)]

        def idx_wait(descs):
            for d in descs:
                d.wait()

        da = idx_load(qof(0), bufa)

        def pair(k, _):
            it0 = 2 * k

            @pl.when(qof(it0) < ntasks)
            def _():
                idx_wait(da)

                @pl.when(qof(it0 + 1) < ntasks)
                def _():
                    idx_wait(idx_load(qof(it0 + 1), bufb))

                dsc_a = _gather_chunk_start(p0, p1, p2, rp, bufa, sem)

                @pl.when(qof(it0 + 1) < ntasks)
                def _():
                    dsc_b = _gather_chunk_start(p0, p1, p2, rp, bufb, sem)

                    @pl.when(qof(it0 + 2) < ntasks)
                    def _():
                        idx_load(qof(it0 + 2), bufa)

                    _gather_chunk_finish(qof(it0), dsc_a, bufa, cv, dr,
                                         s_out, dr_out)
                    _gather_chunk_finish(qof(it0 + 1), dsc_b, bufb, cv, dr,
                                         s_out, dr_out)

                @pl.when(qof(it0 + 1) >= ntasks)
                def _():
                    _gather_chunk_finish(qof(it0), dsc_a, bufa, cv, dr,
                                         s_out, dr_out)

            return 0

        lax.fori_loop(0, -(-nit // 2), pair, 0, unroll=False)

    return body


def _gather_stage(p0, p1, p2, rp, coords_t, cat4):
    mesh = plsc.VectorSubcoreMesh(core_axis_name="c", subcore_axis_name="s")
    f32 = jnp.float32
    i32 = jnp.int32
    ne = cat4.shape[0] // 4

    def gbufs():
        return ([pltpu.VMEM((_GCH,), i32) for _ in range(4)]
                + [pltpu.VMEM((_GCH, 128), f32) for _ in range(4)])

    kern = functools.partial(
        pl.kernel,
        out_type=[
            jax.ShapeDtypeStruct((ne, 128), f32),
            jax.ShapeDtypeStruct((ne * 16,), f32),
        ],
        mesh=mesh,
        scratch_types=[
            gbufs(),
            gbufs(),
            pltpu.VMEM((3 * _N,), f32),
            pltpu.VMEM((_GCH * 16,), f32),
            pltpu.SemaphoreType.DMA,
            pltpu.SemaphoreType.DMA,
        ],
        compiler_params=pltpu.CompilerParams(needs_layout_passes=False),
    )(_make_gather_body(ne))
    return kern(p0, p1, p2, rp, coords_t, cat4)


_BE = 2000


def _leaky(x):
    return jnp.where(x >= 0, x, 0.001 * x)


def _mlp_body(s_ref, dr_ref, b0t_ref, wdl_ref, w1h_ref, b1_ref,
              w2h_ref, b2_ref, w3_ref, b3_ref, pay_ref):
    s = s_ref[...]
    dr = dr_ref[...]
    dl2 = jnp.sum(dr * dr, axis=1, keepdims=True)
    dl = jnp.sqrt(jnp.maximum(dl2, 1e-12))
    dh = dr * (1.0 / dl)
    base = s + dl * wdl_ref[...]
    u = []
    for ti in range(_T):
        h = _leaky(base + b0t_ref[ti:ti + 1, :])
        h = _leaky(jnp.dot(h, w1h_ref[...],
                           preferred_element_type=jnp.float32) + b1_ref[...])
        h = _leaky(jnp.dot(h, w2h_ref[...],
                           preferred_element_type=jnp.float32) + b2_ref[...])
        d = (jnp.dot(h, w3_ref[...],
                     preferred_element_type=jnp.float32) + b3_ref[...])
        u0 = (-0.5 * d[:, 0:1]) * dh
        u2 = (0.5 * d[:, 1:2]) * dh
        u.append((u0, u2))
    pay_ref[0] = jnp.concatenate([u[0][0][:, 0:4], u[1][0][:, 0:4]], axis=1)
    pay_ref[1] = jnp.concatenate([u[0][1][:, 0:4], u[1][1][:, 0:4]], axis=1)


def _mlp_stage(s, dr, b0t, wdl, w1, b1, w2, b2, w3p, b3p):
    ne = s.shape[0]
    return pl.pallas_call(
        _mlp_body,
        grid=(ne // _BE,),
        in_specs=[
            pl.BlockSpec((_BE, 128), lambda i: (i, 0)),
            pl.BlockSpec((_BE, 16), lambda i: (i, 0)),
            pl.BlockSpec((2, 128), lambda i: (0, 0)),
            pl.BlockSpec((1, 128), lambda i: (0, 0)),
            pl.BlockSpec((128, 128), lambda i: (0, 0)),
            pl.BlockSpec((1, 128), lambda i: (0, 0)),
            pl.BlockSpec((128, 128), lambda i: (0, 0)),
            pl.BlockSpec((1, 128), lambda i: (0, 0)),
            pl.BlockSpec((128, 8), lambda i: (0, 0)),
            pl.BlockSpec((1, 8), lambda i: (0, 0)),
        ],
        out_specs=pl.BlockSpec((2, _BE, 8), lambda i: (0, i, 0)),
        out_shape=jax.ShapeDtypeStruct((2, ne, 8), jnp.float32),
    )(s, dr, b0t, wdl, w1, b1, w2, b2, w3p, b3p)


_ACC = _N * 8
_ZLEN = _ACC // _NS


def _make_scatter_body(ntasks):
    def body(pay2d, idx2d, out, idx_v, pay_v, stage_v, acc, lsem, ssem):
        c = lax.axis_index("c")
        s = lax.axis_index("s")
        wid = s * _NC + c

        def zvec(i, _):
            stage_v[pl.ds(i * 16, 16)] = jnp.zeros((16,), jnp.float32)
            return 0

        lax.fori_loop(0, _ZLEN // 16, zvec, 0, unroll=False)
        if _ZLEN % 16:
            stage_v[pl.ds(_ZLEN - 16, 16)] = jnp.zeros((16,), jnp.float32)
        pltpu.sync_copy(stage_v, acc.at[pl.ds(s * _ZLEN, _ZLEN)])
        plsc.subcore_barrier()

        def task(it, _):
            q = wid + _NW * it

            @pl.when(q < ntasks)
            def _():
                r = q * 8
                di = pltpu.async_copy(idx2d.at[pl.ds(r, 8)], idx_v, lsem)
                dp = pltpu.async_copy(pay2d.at[pl.ds(r, 8)], pay_v, lsem)
                di.wait()
                dp.wait()
                descs = [pltpu.async_copy(pay_v.at[j], acc.at[idx_v.at[j]],
                                          ssem, add=True) for j in range(8)]
                for d in descs:
                    d.wait()

            return 0

        lax.fori_loop(0, (ntasks + _NW - 1) // _NW, task, 0, unroll=False)
        plsc.subcore_barrier()
        pltpu.sync_copy(acc.at[pl.ds(s * _ZLEN, _ZLEN)], stage_v)
        pltpu.sync_copy(stage_v, out.at[pl.ds(c * _ACC + s * _ZLEN, _ZLEN)])

    return body


def _scatter_stage(pay2d, idx2d):
    mesh = plsc.VectorSubcoreMesh(core_axis_name="c", subcore_axis_name="s")
    f32 = jnp.float32
    kern = functools.partial(
        pl.kernel,
        out_type=jax.ShapeDtypeStruct((_NC * _ACC,), f32),
        mesh=mesh,
        scratch_types=[
            pltpu.VMEM((8, 128), jnp.int32),
            pltpu.VMEM((8, 128), f32),
            pltpu.VMEM((_ZLEN,), f32),
            pltpu.VMEM_SHARED((_ACC,), f32),
            pltpu.SemaphoreType.DMA,
            pltpu.SemaphoreType.DMA,
        ],
        compiler_params=pltpu.CompilerParams(needs_layout_passes=False),
    )(_make_scatter_body(pay2d.shape[0] // 8))
    return kern(pay2d, idx2d)


def _combine_body(x_ref, a_ref, o_ref):
    acc = a_ref[...]
    for i in range(x_ref.shape[0]):
        acc = acc + x_ref[i]
    o_ref[...] = acc


def _combine_stage(partials, ans_pad):
    return pl.pallas_call(
        _combine_body,
        out_shape=jax.ShapeDtypeStruct((625, 128), jnp.float32),
    )(partials, ans_pad)


def kernel(coords, angles, encoded, t, answer, ring_table,
           W0, b0, W1, b1, W2, b2, W3, b3):
    assert coords.shape == (_N, 3) and angles.shape == (_E, 4)
    ring_norm = 1.0 / np.sqrt(float(_RING_DIM))

    x = jnp.concatenate(
        [encoded, jnp.pad(ring_table, ((0, 0), (0, 16 - _RING_DIM)))], axis=1)
    wall = jnp.zeros((144, 512), jnp.float32)
    wall = wall.at[0:128, 0:384].set(W0[0:384].reshape(3, 128, 128)
                                     .transpose(1, 0, 2).reshape(128, 384))
    wall = wall.at[128:128 + _RING_DIM, 384:512].set(W0[384:384 + _RING_DIM] * ring_norm)
    coords_t = coords.T.reshape(3 * _N)
    b0t = b0[None, :] + t[:, None] * W0[384 + _RING_DIM][None, :]
    wdl = W0[384 + _RING_DIM + 1][None, :]
    w3p = jnp.pad(W3, ((0, 0), (0, 6)))
    b3p = jnp.pad(b3, (0, 6))[None, :]

    a0 = angles[:, 0]
    a1 = angles[:, 1]
    a2 = angles[:, 2]
    ring = angles[:, 3]
    cat4 = jnp.concatenate([a0, a1, a2, ring])

    srows = 2 * _E * 8 // 128
    cat16 = jnp.concatenate([a0, a2]).reshape(srows, 16)
    idx2d = (jnp.repeat(cat16 * 8, 8, axis=1)
             + jnp.tile(jnp.arange(8, dtype=jnp.int32), 16)[None, :])

    p0, p1, p2, rp = _project_tables(x, wall)
    s, dr = _gather_stage(p0, p1, p2, rp, coords_t, cat4)
    dr = dr.reshape(_E, 16)
    pay = _mlp_stage(s, dr, b0t, wdl, W1, b1[None, :], W2, b2[None, :],
                     w3p, b3p)
    part = _scatter_stage(pay.reshape(srows, 128), idx2d)

    ans_pad = jnp.pad(answer, ((0, 0), (0, 0), (0, 1))).reshape(625, 128)
    comb = _combine_stage(part.reshape(_NC, 625, 128), ans_pad)
    return comb.reshape(_N, _T, 4)[:, :, :3]


# speedup vs baseline: 1.3748x; 1.0011x over previous
import functools

import jax
import jax.numpy as jnp
import numpy as np
from jax import lax
from jax.experimental import pallas as pl
from jax.experimental.pallas import tpu as pltpu
from jax.experimental.pallas import tpu_sc as plsc

_NC = 2
_NS = 16
_NW = _NC * _NS

_N = 10000
_E = 160000
_D = 128
_T = 2
_RING_DIM = 10
_HID = 128


_A_ROWS = 1000


def _proj_body(x_ref, w_ref, p0_ref, p1_ref, p2_ref, rp_ref):
    acc = jnp.dot(x_ref[...], w_ref[...], preferred_element_type=jnp.float32)
    p0_ref[...] = acc[:, 0:128]
    p1_ref[...] = acc[:, 128:256]
    p2_ref[...] = acc[:, 256:384]
    rp_ref[...] = acc[:, 384:512]


def _project_tables(x, wall):
    n = x.shape[0]
    grid = n // _A_ROWS
    out = jax.ShapeDtypeStruct((n, 128), jnp.float32)
    return pl.pallas_call(
        _proj_body,
        grid=(grid,),
        in_specs=[
            pl.BlockSpec((_A_ROWS, 144), lambda i: (i, 0)),
            pl.BlockSpec((144, 512), lambda i: (0, 0)),
        ],
        out_specs=[pl.BlockSpec((_A_ROWS, 128), lambda i: (i, 0))] * 4,
        out_shape=[out, out, out, out],
    )(x, wall)


_GCH = 64
_GTASKS = _E // _GCH
_GIT = -(-_GTASKS // _NW)


def _gather_chunk_start(p0, p1, p2, rp, bufs, sem):
    i0, i1, i2, i3, g0, g1, g2, g3 = bufs
    return (pltpu.async_copy(p0.at[i0], g0, sem),
            pltpu.async_copy(p1.at[i1], g1, sem),
            pltpu.async_copy(p2.at[i2], g2, sem),
            pltpu.async_copy(rp.at[i3], g3, sem))


def _gather_chunk_finish(q, descs, bufs, cv, dr, s_out, dr_out):
    i0, i1, i2, i3, g0, g1, g2, g3 = bufs
    base = q * _GCH
    lanes = lax.iota(jnp.int32, 16)
    for g in range(_GCH // 16):
        off = g * 16
        n0 = i0[pl.ds(off, 16)]
        n2 = i2[pl.ds(off, 16)]
        rows = (lanes + off) * 16
        for k in range(3):
            x0 = plsc.load_gather(cv, [n0 + k * _N])
            x2 = plsc.load_gather(cv, [n2 + k * _N])
            plsc.store_scatter(dr, [rows + k], x0 - x2)
    pltpu.sync_copy(dr, dr_out.at[pl.ds(base * 16, _GCH * 16)])
    for d in descs:
        d.wait()

    def row(i, _):
        for j in range(8):
            sl = pl.ds(j * 16, 16)
            g0[i, sl] = g0[i, sl] + g1[i, sl] + g2[i, sl] + g3[i, sl]
        return 0

    lax.fori_loop(0, _GCH, row, 0, unroll=False)
    pltpu.sync_copy(g0, s_out.at[pl.ds(base, _GCH)])


def _make_gather_body(ne):
    ntasks = ne // _GCH
    nit = -(-ntasks // _NW)

    def body(p0, p1, p2, rp, coords_t, cat4,
             s_out, dr_out,
             bufa, bufb, cv, dr, sem, isem):
        wid = lax.axis_index("s") * _NC + lax.axis_index("c")

        pltpu.sync_copy(coords_t, cv)

        def zrow(i, _):
            dr[pl.ds(i * 16, 16)] = jnp.zeros((16,), jnp.float32)
            return 0

        lax.fori_loop(0, _GCH, zrow, 0, unroll=False)

        def qof(it):
            return wid + _NW * it

        def idx_load(q, bufs):
            return [pltpu.async_copy(
                cat4.at[pl.ds(k * ne + q * _GCH, _GCH)], bufs[k], isem)
                for k in range(4)]

        def idx_wait(descs):
            for d in descs:
                d.wait()

        da = idx_load(qof(0), bufa)

        def pair(k, _):
            it0 = 2 * k

            @pl.when(qof(it0) < ntasks)
            def _():
                idx_wait(da)

                @pl.when(qof(it0 + 1) < ntasks)
                def _():
                    idx_wait(idx_load(qof(it0 + 1), bufb))

                dsc_a = _gather_chunk_start(p0, p1, p2, rp, bufa, sem)

                @pl.when(qof(it0 + 1) < ntasks)
                def _():
                    dsc_b = _gather_chunk_start(p0, p1, p2, rp, bufb, sem)

                    _gather_chunk_finish(qof(it0), dsc_a, bufa, cv, dr,
                                         s_out, dr_out)

                    @pl.when(qof(it0 + 2) < ntasks)
                    def _():
                        idx_load(qof(it0 + 2), bufa)

                    _gather_chunk_finish(qof(it0 + 1), dsc_b, bufb, cv, dr,
                                         s_out, dr_out)

                @pl.when(qof(it0 + 1) >= ntasks)
                def _():
                    _gather_chunk_finish(qof(it0), dsc_a, bufa, cv, dr,
                                         s_out, dr_out)

            return 0

        lax.fori_loop(0, -(-nit // 2), pair, 0, unroll=False)

    return body


def _gather_stage(p0, p1, p2, rp, coords_t, cat4):
    mesh = plsc.VectorSubcoreMesh(core_axis_name="c", subcore_axis_name="s")
    f32 = jnp.float32
    i32 = jnp.int32
    ne = cat4.shape[0] // 4

    def gbufs():
        return ([pltpu.VMEM((_GCH,), i32) for _ in range(4)]
                + [pltpu.VMEM((_GCH, 128), f32) for _ in range(4)])

    kern = functools.partial(
        pl.kernel,
        out_type=[
            jax.ShapeDtypeStruct((ne, 128), f32),
            jax.ShapeDtypeStruct((ne * 16,), f32),
        ],
        mesh=mesh,
        scratch_types=[
            gbufs(),
            gbufs(),
            pltpu.VMEM((3 * _N,), f32),
            pltpu.VMEM((_GCH * 16,), f32),
            pltpu.SemaphoreType.DMA,
            pltpu.SemaphoreType.DMA,
        ],
        compiler_params=pltpu.CompilerParams(needs_layout_passes=False),
    )(_make_gather_body(ne))
    return kern(p0, p1, p2, rp, coords_t, cat4)


_BE = 2000


def _leaky(x):
    return jnp.where(x >= 0, x, 0.001 * x)


def _mlp_body(s_ref, dr_ref, b0t_ref, wdl_ref, w1h_ref, b1_ref,
              w2h_ref, b2_ref, w3_ref, b3_ref, pay_ref):
    s = s_ref[...]
    dr = dr_ref[...]
    dl2 = jnp.sum(dr * dr, axis=1, keepdims=True)
    dl = jnp.sqrt(jnp.maximum(dl2, 1e-12))
    dh = dr * (1.0 / dl)
    base = s + dl * wdl_ref[...]
    u = []
    for ti in range(_T):
        h = _leaky(base + b0t_ref[ti:ti + 1, :])
        h = _leaky(jnp.dot(h, w1h_ref[...],
                           preferred_element_type=jnp.float32) + b1_ref[...])
        h = _leaky(jnp.dot(h, w2h_ref[...],
                           preferred_element_type=jnp.float32) + b2_ref[...])
        d = (jnp.dot(h, w3_ref[...],
                     preferred_element_type=jnp.float32) + b3_ref[...])
        u0 = (-0.5 * d[:, 0:1]) * dh
        u2 = (0.5 * d[:, 1:2]) * dh
        u.append((u0, u2))
    pay_ref[0] = jnp.concatenate([u[0][0][:, 0:4], u[1][0][:, 0:4]], axis=1)
    pay_ref[1] = jnp.concatenate([u[0][1][:, 0:4], u[1][1][:, 0:4]], axis=1)


def _mlp_stage(s, dr, b0t, wdl, w1, b1, w2, b2, w3p, b3p):
    ne = s.shape[0]
    return pl.pallas_call(
        _mlp_body,
        grid=(ne // _BE,),
        in_specs=[
            pl.BlockSpec((_BE, 128), lambda i: (i, 0)),
            pl.BlockSpec((_BE, 16), lambda i: (i, 0)),
            pl.BlockSpec((2, 128), lambda i: (0, 0)),
            pl.BlockSpec((1, 128), lambda i: (0, 0)),
            pl.BlockSpec((128, 128), lambda i: (0, 0)),
            pl.BlockSpec((1, 128), lambda i: (0, 0)),
            pl.BlockSpec((128, 128), lambda i: (0, 0)),
            pl.BlockSpec((1, 128), lambda i: (0, 0)),
            pl.BlockSpec((128, 8), lambda i: (0, 0)),
            pl.BlockSpec((1, 8), lambda i: (0, 0)),
        ],
        out_specs=pl.BlockSpec((2, _BE, 8), lambda i: (0, i, 0)),
        out_shape=jax.ShapeDtypeStruct((2, ne, 8), jnp.float32),
    )(s, dr, b0t, wdl, w1, b1, w2, b2, w3p, b3p)


_ACC = _N * 8
_ZLEN = _ACC // _NS


def _make_scatter_body(ntasks):
    def body(pay2d, idx2d, out, idx_v, pay_v, stage_v, acc, lsem, ssem):
        c = lax.axis_index("c")
        s = lax.axis_index("s")
        wid = s * _NC + c

        def zvec(i, _):
            stage_v[pl.ds(i * 16, 16)] = jnp.zeros((16,), jnp.float32)
            return 0

        lax.fori_loop(0, _ZLEN // 16, zvec, 0, unroll=False)
        if _ZLEN % 16:
            stage_v[pl.ds(_ZLEN - 16, 16)] = jnp.zeros((16,), jnp.float32)
        pltpu.sync_copy(stage_v, acc.at[pl.ds(s * _ZLEN, _ZLEN)])
        plsc.subcore_barrier()

        def task(it, _):
            q = wid + _NW * it

            @pl.when(q < ntasks)
            def _():
                r = q * 8
                di = pltpu.async_copy(idx2d.at[pl.ds(r, 8)], idx_v, lsem)
                dp = pltpu.async_copy(pay2d.at[pl.ds(r, 8)], pay_v, lsem)
                di.wait()
                dp.wait()
                descs = [pltpu.async_copy(pay_v.at[j], acc.at[idx_v.at[j]],
                                          ssem, add=True) for j in range(8)]
                for d in descs:
                    d.wait()

            return 0

        lax.fori_loop(0, (ntasks + _NW - 1) // _NW, task, 0, unroll=False)
        plsc.subcore_barrier()
        pltpu.sync_copy(acc.at[pl.ds(s * _ZLEN, _ZLEN)], stage_v)
        pltpu.sync_copy(stage_v, out.at[pl.ds(c * _ACC + s * _ZLEN, _ZLEN)])

    return body


def _scatter_stage(pay2d, idx2d):
    mesh = plsc.VectorSubcoreMesh(core_axis_name="c", subcore_axis_name="s")
    f32 = jnp.float32
    kern = functools.partial(
        pl.kernel,
        out_type=jax.ShapeDtypeStruct((_NC * _ACC,), f32),
        mesh=mesh,
        scratch_types=[
            pltpu.VMEM((8, 128), jnp.int32),
            pltpu.VMEM((8, 128), f32),
            pltpu.VMEM((_ZLEN,), f32),
            pltpu.VMEM_SHARED((_ACC,), f32),
            pltpu.SemaphoreType.DMA,
            pltpu.SemaphoreType.DMA,
        ],
        compiler_params=pltpu.CompilerParams(needs_layout_passes=False),
    )(_make_scatter_body(pay2d.shape[0] // 8))
    return kern(pay2d, idx2d)


def _combine_body(x_ref, a_ref, o_ref):
    acc = a_ref[...]
    for i in range(x_ref.shape[0]):
        acc = acc + x_ref[i]
    o_ref[...] = acc


def _combine_stage(partials, ans_pad):
    return pl.pallas_call(
        _combine_body,
        out_shape=jax.ShapeDtypeStruct((625, 128), jnp.float32),
    )(partials, ans_pad)


def kernel(coords, angles, encoded, t, answer, ring_table,
           W0, b0, W1, b1, W2, b2, W3, b3):
    assert coords.shape == (_N, 3) and angles.shape == (_E, 4)
    ring_norm = 1.0 / np.sqrt(float(_RING_DIM))

    x = jnp.concatenate(
        [encoded, jnp.pad(ring_table, ((0, 0), (0, 16 - _RING_DIM)))], axis=1)
    wall = jnp.zeros((144, 512), jnp.float32)
    wall = wall.at[0:128, 0:384].set(W0[0:384].reshape(3, 128, 128)
                                     .transpose(1, 0, 2).reshape(128, 384))
    wall = wall.at[128:128 + _RING_DIM, 384:512].set(W0[384:384 + _RING_DIM] * ring_norm)
    coords_t = coords.T.reshape(3 * _N)
    b0t = b0[None, :] + t[:, None] * W0[384 + _RING_DIM][None, :]
    wdl = W0[384 + _RING_DIM + 1][None, :]
    w3p = jnp.pad(W3, ((0, 0), (0, 6)))
    b3p = jnp.pad(b3, (0, 6))[None, :]

    a0 = angles[:, 0]
    a1 = angles[:, 1]
    a2 = angles[:, 2]
    ring = angles[:, 3]
    cat4 = jnp.concatenate([a0, a1, a2, ring])

    srows = 2 * _E * 8 // 128
    cat16 = jnp.concatenate([a0, a2]).reshape(srows, 16)
    idx2d = (jnp.repeat(cat16 * 8, 8, axis=1)
             + jnp.tile(jnp.arange(8, dtype=jnp.int32), 16)[None, :])

    p0, p1, p2, rp = _project_tables(x, wall)
    s, dr = _gather_stage(p0, p1, p2, rp, coords_t, cat4)
    dr = dr.reshape(_E, 16)
    pay = _mlp_stage(s, dr, b0t, wdl, W1, b1[None, :], W2, b2[None, :],
                     w3p, b3p)
    part = _scatter_stage(pay.reshape(srows, 128), idx2d)

    ans_pad = jnp.pad(answer, ((0, 0), (0, 0), (0, 1))).reshape(625, 128)
    comb = _combine_stage(part.reshape(_NC, 625, 128), ans_pad)
    return comb.reshape(_N, _T, 4)[:, :, :3]


# speedup vs baseline: 1.3792x; 1.0032x over previous
import functools

import jax
import jax.numpy as jnp
import numpy as np
from jax import lax
from jax.experimental import pallas as pl
from jax.experimental.pallas import tpu as pltpu
from jax.experimental.pallas import tpu_sc as plsc

_NC = 2
_NS = 16
_NW = _NC * _NS

_N = 10000
_E = 160000
_D = 128
_T = 2
_RING_DIM = 10
_HID = 128


_A_ROWS = 1000


def _proj_body(x_ref, w_ref, p0_ref, p1_ref, p2_ref, rp_ref):
    acc = jnp.dot(x_ref[...], w_ref[...], preferred_element_type=jnp.float32)
    p0_ref[...] = acc[:, 0:128]
    p1_ref[...] = acc[:, 128:256]
    p2_ref[...] = acc[:, 256:384]
    rp_ref[...] = acc[:, 384:512]


def _project_tables(x, wall):
    n = x.shape[0]
    grid = n // _A_ROWS
    out = jax.ShapeDtypeStruct((n, 128), jnp.float32)
    return pl.pallas_call(
        _proj_body,
        grid=(grid,),
        in_specs=[
            pl.BlockSpec((_A_ROWS, 144), lambda i: (i, 0)),
            pl.BlockSpec((144, 512), lambda i: (0, 0)),
        ],
        out_specs=[pl.BlockSpec((_A_ROWS, 128), lambda i: (i, 0))] * 4,
        out_shape=[out, out, out, out],
    )(x, wall)


_GCH = 64
_GTASKS = _E // _GCH
_GIT = -(-_GTASKS // _NW)


def _gather_chunk_start(p0, p1, p2, rp, bufs, sem):
    i0, i1, i2, i3, g0, g1, g2, g3 = bufs
    return (pltpu.async_copy(p0.at[i0], g0, sem),
            pltpu.async_copy(p1.at[i1], g1, sem),
            pltpu.async_copy(p2.at[i2], g2, sem),
            pltpu.async_copy(rp.at[i3], g3, sem))


def _gather_chunk_finish(q, descs, bufs, cv, dr, s_out, dr_out):
    i0, i1, i2, i3, g0, g1, g2, g3 = bufs
    base = q * _GCH
    lanes = lax.iota(jnp.int32, 16)
    for g in range(_GCH // 16):
        off = g * 16
        n0 = i0[pl.ds(off, 16)]
        n2 = i2[pl.ds(off, 16)]
        rows = (lanes + off) * 16
        for k in range(3):
            x0 = plsc.load_gather(cv, [n0 + k * _N])
            x2 = plsc.load_gather(cv, [n2 + k * _N])
            plsc.store_scatter(dr, [rows + k], x0 - x2)
    pltpu.sync_copy(dr, dr_out.at[pl.ds(base * 16, _GCH * 16)])
    for d in descs:
        d.wait()

    def row(i, _):
        for j in range(8):
            sl = pl.ds(j * 16, 16)
            g0[i, sl] = g0[i, sl] + g1[i, sl] + g2[i, sl] + g3[i, sl]
        return 0

    lax.fori_loop(0, _GCH, row, 0, unroll=False)
    pltpu.sync_copy(g0, s_out.at[pl.ds(base, _GCH)])


def _make_gather_body(ne):
    ntasks = ne // _GCH
    nit = -(-ntasks // _NW)

    def body(p0, p1, p2, rp, coords_t, cat4,
             s_out, dr_out,
             bufa, bufb, cv, dr, sem, isem):
        wid = lax.axis_index("s") * _NC + lax.axis_index("c")

        pltpu.sync_copy(coords_t, cv)

        def zrow(i, _):
            dr[pl.ds(i * 16, 16)] = jnp.zeros((16,), jnp.float32)
            return 0

        lax.fori_loop(0, _GCH, zrow, 0, unroll=False)

        def qof(it):
            return wid + _NW * it

        def idx_load(q, bufs):
            return [pltpu.async_copy(
                cat4.at[pl.ds(k * ne + q * _GCH, _GCH)], bufs[k], isem)
                for k in range(4)]

        def idx_wait(descs):
            for d in descs:
                d.wait()

        da = idx_load(qof(0), bufa)

        def pair(k, _):
            it0 = 2 * k

            @pl.when(qof(it0) < ntasks)
            def _():
                idx_wait(da)

                @pl.when(qof(it0 + 1) < ntasks)
                def _():
                    idx_wait(idx_load(qof(it0 + 1), bufb))

                dsc_a = _gather_chunk_start(p0, p1, p2, rp, bufa, sem)

                @pl.when(qof(it0 + 1) < ntasks)
                def _():
                    dsc_b = _gather_chunk_start(p0, p1, p2, rp, bufb, sem)

                    _gather_chunk_finish(qof(it0), dsc_a, bufa, cv, dr,
                                         s_out, dr_out)

                    @pl.when(qof(it0 + 2) < ntasks)
                    def _():
                        idx_load(qof(it0 + 2), bufa)

                    _gather_chunk_finish(qof(it0 + 1), dsc_b, bufb, cv, dr,
                                         s_out, dr_out)

                @pl.when(qof(it0 + 1) >= ntasks)
                def _():
                    _gather_chunk_finish(qof(it0), dsc_a, bufa, cv, dr,
                                         s_out, dr_out)

            return 0

        lax.fori_loop(0, -(-nit // 2), pair, 0, unroll=False)

    return body


def _gather_stage(p0, p1, p2, rp, coords_t, cat4):
    mesh = plsc.VectorSubcoreMesh(core_axis_name="c", subcore_axis_name="s")
    f32 = jnp.float32
    i32 = jnp.int32
    ne = cat4.shape[0] // 4

    def gbufs():
        return ([pltpu.VMEM((_GCH,), i32) for _ in range(4)]
                + [pltpu.VMEM((_GCH, 128), f32) for _ in range(4)])

    kern = functools.partial(
        pl.kernel,
        out_type=[
            jax.ShapeDtypeStruct((ne, 128), f32),
            jax.ShapeDtypeStruct((ne * 16,), f32),
        ],
        mesh=mesh,
        scratch_types=[
            gbufs(),
            gbufs(),
            pltpu.VMEM((3 * _N,), f32),
            pltpu.VMEM((_GCH * 16,), f32),
            pltpu.SemaphoreType.DMA,
            pltpu.SemaphoreType.DMA,
        ],
        compiler_params=pltpu.CompilerParams(needs_layout_passes=False),
    )(_make_gather_body(ne))
    return kern(p0, p1, p2, rp, coords_t, cat4)


_BE = 3200


def _leaky(x):
    return jnp.where(x >= 0, x, 0.001 * x)


def _mlp_body(s_ref, dr_ref, b0t_ref, wdl_ref, w1h_ref, b1_ref,
              w2h_ref, b2_ref, w3_ref, b3_ref, pay_ref):
    s = s_ref[...]
    dr = dr_ref[...]
    dl2 = jnp.sum(dr * dr, axis=1, keepdims=True)
    dl = jnp.sqrt(jnp.maximum(dl2, 1e-12))
    dh = dr * (1.0 / dl)
    base = s + dl * wdl_ref[...]
    u = []
    for ti in range(_T):
        h = _leaky(base + b0t_ref[ti:ti + 1, :])
        h = _leaky(jnp.dot(h, w1h_ref[...],
                           preferred_element_type=jnp.float32) + b1_ref[...])
        h = _leaky(jnp.dot(h, w2h_ref[...],
                           preferred_element_type=jnp.float32) + b2_ref[...])
        d = (jnp.dot(h, w3_ref[...],
                     preferred_element_type=jnp.float32) + b3_ref[...])
        u0 = (-0.5 * d[:, 0:1]) * dh
        u2 = (0.5 * d[:, 1:2]) * dh
        u.append((u0, u2))
    pay_ref[0] = jnp.concatenate([u[0][0][:, 0:4], u[1][0][:, 0:4]], axis=1)
    pay_ref[1] = jnp.concatenate([u[0][1][:, 0:4], u[1][1][:, 0:4]], axis=1)


def _mlp_stage(s, dr, b0t, wdl, w1, b1, w2, b2, w3p, b3p):
    ne = s.shape[0]
    return pl.pallas_call(
        _mlp_body,
        grid=(ne // _BE,),
        in_specs=[
            pl.BlockSpec((_BE, 128), lambda i: (i, 0)),
            pl.BlockSpec((_BE, 16), lambda i: (i, 0)),
            pl.BlockSpec((2, 128), lambda i: (0, 0)),
            pl.BlockSpec((1, 128), lambda i: (0, 0)),
            pl.BlockSpec((128, 128), lambda i: (0, 0)),
            pl.BlockSpec((1, 128), lambda i: (0, 0)),
            pl.BlockSpec((128, 128), lambda i: (0, 0)),
            pl.BlockSpec((1, 128), lambda i: (0, 0)),
            pl.BlockSpec((128, 8), lambda i: (0, 0)),
            pl.BlockSpec((1, 8), lambda i: (0, 0)),
        ],
        out_specs=pl.BlockSpec((2, _BE, 8), lambda i: (0, i, 0)),
        out_shape=jax.ShapeDtypeStruct((2, ne, 8), jnp.float32),
    )(s, dr, b0t, wdl, w1, b1, w2, b2, w3p, b3p)


_ACC = _N * 8
_ZLEN = _ACC // _NS


def _make_scatter_body(ntasks):
    def body(pay2d, idx2d, out, idx_v, pay_v, stage_v, acc, lsem, ssem):
        c = lax.axis_index("c")
        s = lax.axis_index("s")
        wid = s * _NC + c

        def zvec(i, _):
            stage_v[pl.ds(i * 16, 16)] = jnp.zeros((16,), jnp.float32)
            return 0

        lax.fori_loop(0, _ZLEN // 16, zvec, 0, unroll=False)
        if _ZLEN % 16:
            stage_v[pl.ds(_ZLEN - 16, 16)] = jnp.zeros((16,), jnp.float32)
        pltpu.sync_copy(stage_v, acc.at[pl.ds(s * _ZLEN, _ZLEN)])
        plsc.subcore_barrier()

        def task(it, _):
            q = wid + _NW * it

            @pl.when(q < ntasks)
            def _():
                r = q * 8
                di = pltpu.async_copy(idx2d.at[pl.ds(r, 8)], idx_v, lsem)
                dp = pltpu.async_copy(pay2d.at[pl.ds(r, 8)], pay_v, lsem)
                di.wait()
                dp.wait()
                descs = [pltpu.async_copy(pay_v.at[j], acc.at[idx_v.at[j]],
                                          ssem, add=True) for j in range(8)]
                for d in descs:
                    d.wait()

            return 0

        lax.fori_loop(0, (ntasks + _NW - 1) // _NW, task, 0, unroll=False)
        plsc.subcore_barrier()
        pltpu.sync_copy(acc.at[pl.ds(s * _ZLEN, _ZLEN)], stage_v)
        pltpu.sync_copy(stage_v, out.at[pl.ds(c * _ACC + s * _ZLEN, _ZLEN)])

    return body


def _scatter_stage(pay2d, idx2d):
    mesh = plsc.VectorSubcoreMesh(core_axis_name="c", subcore_axis_name="s")
    f32 = jnp.float32
    kern = functools.partial(
        pl.kernel,
        out_type=jax.ShapeDtypeStruct((_NC * _ACC,), f32),
        mesh=mesh,
        scratch_types=[
            pltpu.VMEM((8, 128), jnp.int32),
            pltpu.VMEM((8, 128), f32),
            pltpu.VMEM((_ZLEN,), f32),
            pltpu.VMEM_SHARED((_ACC,), f32),
            pltpu.SemaphoreType.DMA,
            pltpu.SemaphoreType.DMA,
        ],
        compiler_params=pltpu.CompilerParams(needs_layout_passes=False),
    )(_make_scatter_body(pay2d.shape[0] // 8))
    return kern(pay2d, idx2d)


def _combine_body(x_ref, a_ref, o_ref):
    acc = a_ref[...]
    for i in range(x_ref.shape[0]):
        acc = acc + x_ref[i]
    o_ref[...] = acc


def _combine_stage(partials, ans_pad):
    return pl.pallas_call(
        _combine_body,
        out_shape=jax.ShapeDtypeStruct((625, 128), jnp.float32),
    )(partials, ans_pad)


def kernel(coords, angles, encoded, t, answer, ring_table,
           W0, b0, W1, b1, W2, b2, W3, b3):
    assert coords.shape == (_N, 3) and angles.shape == (_E, 4)
    ring_norm = 1.0 / np.sqrt(float(_RING_DIM))

    x = jnp.concatenate(
        [encoded, jnp.pad(ring_table, ((0, 0), (0, 16 - _RING_DIM)))], axis=1)
    wall = jnp.zeros((144, 512), jnp.float32)
    wall = wall.at[0:128, 0:384].set(W0[0:384].reshape(3, 128, 128)
                                     .transpose(1, 0, 2).reshape(128, 384))
    wall = wall.at[128:128 + _RING_DIM, 384:512].set(W0[384:384 + _RING_DIM] * ring_norm)
    coords_t = coords.T.reshape(3 * _N)
    b0t = b0[None, :] + t[:, None] * W0[384 + _RING_DIM][None, :]
    wdl = W0[384 + _RING_DIM + 1][None, :]
    w3p = jnp.pad(W3, ((0, 0), (0, 6)))
    b3p = jnp.pad(b3, (0, 6))[None, :]

    a0 = angles[:, 0]
    a1 = angles[:, 1]
    a2 = angles[:, 2]
    ring = angles[:, 3]
    cat4 = jnp.concatenate([a0, a1, a2, ring])

    srows = 2 * _E * 8 // 128
    cat16 = jnp.concatenate([a0, a2]).reshape(srows, 16)
    idx2d = (jnp.repeat(cat16 * 8, 8, axis=1)
             + jnp.tile(jnp.arange(8, dtype=jnp.int32), 16)[None, :])

    p0, p1, p2, rp = _project_tables(x, wall)
    s, dr = _gather_stage(p0, p1, p2, rp, coords_t, cat4)
    dr = dr.reshape(_E, 16)
    pay = _mlp_stage(s, dr, b0t, wdl, W1, b1[None, :], W2, b2[None, :],
                     w3p, b3p)
    part = _scatter_stage(pay.reshape(srows, 128), idx2d)

    ans_pad = jnp.pad(answer, ((0, 0), (0, 0), (0, 1))).reshape(625, 128)
    comb = _combine_stage(part.reshape(_NC, 625, 128), ans_pad)
    return comb.reshape(_N, _T, 4)[:, :, :3]
